# baseline (speedup 1.0000x reference)
import functools

import jax
import jax.numpy as jnp
from jax import lax
from jax.experimental import pallas as pl
from jax.experimental.pallas import tpu as pltpu
from jax.experimental.pallas import tpu_sc as plsc

ROWS = 32
K = 24
BEAM = 16
VOCAB = 1000000
LANES = 16
OFF = 16
NEG_INF = float("-inf")


def _sc_body(scores_hbm, ids_hbm, tv_hbm, ti_hbm, li_hbm,
             idbuf, fidx, vals, outv, outg, outl, sem):
    b = lax.axis_index("c") * 16 + lax.axis_index("s")

    lane = lax.iota(jnp.int32, LANES)

    pltpu.sync_copy(ids_hbm.at[pl.ds(pl.multiple_of(b * K, 8), K)],
                    idbuf.at[pl.ds(OFF, K)])
    g0 = idbuf[pl.ds(OFF, LANES)]
    g1 = jnp.where(lane < (K - LANES), idbuf[pl.ds(OFF + LANES, LANES)], 0)

    base = b * VOCAB
    fidx[pl.ds(0, LANES)] = g0 + base
    fidx[pl.ds(LANES, LANES)] = g1 + base
    pltpu.async_copy(scores_hbm.at[fidx], vals.at[pl.ds(OFF, 2 * LANES)],
                     sem).wait()

    val0 = vals[pl.ds(OFF, LANES)]
    val1 = jnp.where(lane < (K - LANES), vals[pl.ds(OFF + LANES, LANES)],
                     NEG_INF)

    zero = jnp.zeros((LANES,), jnp.int32)
    lrank = [zero, zero]
    dup = [zero, zero]
    grank = [zero, zero]
    gvec = (g0, g1)
    vvec = (val0, val1)

    for a in range(K):
        splat_a = lane * 0 + (OFF + a)
        bval = plsc.load_gather(vals, [splat_a])
        bg = plsc.load_gather(idbuf, [splat_a])
        for blk in range(2):
            a_lt_i = (lane + blk * LANES) > a
            v = vvec[blk]
            lrank[blk] = lrank[blk] + (
                (bval > v) | ((bval == v) & a_lt_i)).astype(jnp.int32)
            dup[blk] = dup[blk] + (
                (bg == gvec[blk]) & a_lt_i).astype(jnp.int32)

    vval0 = jnp.where(dup[0] == 0, val0, NEG_INF)
    vval1 = jnp.where(dup[1] == 0, val1, NEG_INF)
    vals[pl.ds(OFF, LANES)] = vval0
    vals[pl.ds(OFF + LANES, LANES)] = vval1

    for a in range(K):
        splat_a = lane * 0 + (OFF + a)
        bvv = plsc.load_gather(vals, [splat_a])
        bg = plsc.load_gather(idbuf, [splat_a])
        for blk in range(2):
            a_lt_i = (lane + blk * LANES) > a
            v = vvec[blk]
            g = gvec[blk]
            grank[blk] = grank[blk] + (
                (bvv > v) | ((bvv == v) & ((bg < g) | ((bg == g) & a_lt_i)))
            ).astype(jnp.int32)

    real = (lane < (K - LANES))
    for blk in range(2):
        lane_ok = (lane >= 0) if blk == 0 else real
        gmask = lane_ok & (dup[blk] == 0) & (grank[blk] < BEAM)
        gidx = jnp.minimum(grank[blk], BEAM - 1)
        plsc.store_scatter(outv, [gidx], vvec[blk], mask=gmask)
        plsc.store_scatter(outg, [gidx], gvec[blk], mask=gmask)
        lmask = lane_ok & (lrank[blk] < BEAM)
        lidx = jnp.minimum(lrank[blk], BEAM - 1)
        plsc.store_scatter(outl, [lidx], lane + blk * LANES, mask=lmask)

    out_off = pl.multiple_of(b * BEAM, 8)
    pltpu.sync_copy(outv, tv_hbm.at[pl.ds(out_off, BEAM)])
    pltpu.sync_copy(outg, ti_hbm.at[pl.ds(out_off, BEAM)])
    pltpu.sync_copy(outl, li_hbm.at[pl.ds(out_off, BEAM)])


_sc_call = functools.partial(
    pl.kernel,
    out_type=[
        jax.ShapeDtypeStruct((ROWS * BEAM,), jnp.float32),
        jax.ShapeDtypeStruct((ROWS * BEAM,), jnp.int32),
        jax.ShapeDtypeStruct((ROWS * BEAM,), jnp.int32),
    ],
    mesh=plsc.VectorSubcoreMesh(
        core_axis_name="c", subcore_axis_name="s", num_cores=2,
        num_subcores=16),
    compiler_params=pltpu.CompilerParams(needs_layout_passes=False),
    scratch_types=[
        pltpu.VMEM((OFF + 2 * LANES,), jnp.int32),
        pltpu.VMEM((2 * LANES,), jnp.int32),
        pltpu.VMEM((OFF + 2 * LANES,), jnp.float32),
        pltpu.VMEM((BEAM,), jnp.float32),
        pltpu.VMEM((BEAM,), jnp.int32),
        pltpu.VMEM((BEAM,), jnp.int32),
        pltpu.SemaphoreType.DMA,
    ],
)(_sc_body)


def kernel(weighted_scores, ids):
    scores_flat = weighted_scores.reshape(-1)
    ids_flat = ids.astype(jnp.int32).reshape(-1)
    tv, ti, li = _sc_call(scores_flat, ids_flat)
    return (tv.reshape(ROWS, BEAM), ti.reshape(ROWS, BEAM),
            li.reshape(ROWS, BEAM))


# speedup vs baseline: 90.3452x; 90.3452x over previous
import functools

import jax
import jax.numpy as jnp
from jax import lax
from jax.experimental import pallas as pl
from jax.experimental.pallas import tpu as pltpu
from jax.experimental.pallas import tpu_sc as plsc

ROWS = 32
K = 24
BEAM = 16
VOCAB = 1000000
LANES = 16
NEG_INF = float("-inf")


def _sc_body(scores_hbm, ids_hbm, tv_hbm, ti_hbm, li_hbm,
             idbuf, chunk, outv, outg, outl, sem):
    b = lax.axis_index("c") * 16 + lax.axis_index("s")

    lane = lax.iota(jnp.int32, LANES)

    pltpu.sync_copy(ids_hbm.at[pl.ds(pl.multiple_of(b * K, 8), K)],
                    idbuf.at[pl.ds(0, K)])
    g0 = idbuf[pl.ds(0, LANES)]
    g1 = jnp.where(lane < (K - LANES), idbuf[pl.ds(LANES, LANES)], 0)

    row8 = pl.multiple_of((b >> 3) << 3, 8)
    copies = []
    for j in range(K):
        gj = (g0 if j < LANES else g1)[j % LANES]
        col128 = pl.multiple_of((gj >> 7) << 7, 128)
        copies.append(pltpu.async_copy(
            scores_hbm.at[pl.ds(row8, 8), pl.ds(col128, 128)],
            chunk.at[j + 1], sem))
    for c in copies:
        c.wait()

    rvec = lane * 0 + (b & 7)
    val0 = plsc.load_gather(chunk, [lane + 1, rvec, g0 & 127])
    rawv1 = plsc.load_gather(chunk, [lane + 1 + LANES, rvec, g1 & 127])
    val1 = jnp.where(lane < (K - LANES), rawv1, NEG_INF)

    zero = jnp.zeros((LANES,), jnp.int32)
    lrank = [zero, zero]
    dup = [zero, zero]
    grank = [zero, zero]
    gvec = (g0, g1)
    vvec = (val0, val1)

    for a in range(K):
        bval = vvec[a // LANES][a % LANES]
        bg = gvec[a // LANES][a % LANES]
        for blk in range(2):
            a_lt_i = (lane + blk * LANES) > a
            v = vvec[blk]
            lrank[blk] = lrank[blk] + (
                (bval > v) | ((bval == v) & a_lt_i)).astype(jnp.int32)
            dup[blk] = dup[blk] + (
                (bg == gvec[blk]) & a_lt_i).astype(jnp.int32)

    vval0 = jnp.where(dup[0] == 0, val0, NEG_INF)
    vval1 = jnp.where(dup[1] == 0, val1, NEG_INF)
    wvec = (vval0, vval1)

    for a in range(K):
        bvv = wvec[a // LANES][a % LANES]
        bg = gvec[a // LANES][a % LANES]
        for blk in range(2):
            a_lt_i = (lane + blk * LANES) > a
            v = vvec[blk]
            g = gvec[blk]
            grank[blk] = grank[blk] + (
                (bvv > v) | ((bvv == v) & ((bg < g) | ((bg == g) & a_lt_i)))
            ).astype(jnp.int32)

    real = (lane < (K - LANES))
    for blk in range(2):
        lane_ok = (lane >= 0) if blk == 0 else real
        gmask = lane_ok & (dup[blk] == 0) & (grank[blk] < BEAM)
        gidx = jnp.minimum(grank[blk], BEAM - 1)
        plsc.store_scatter(outv, [gidx], vvec[blk], mask=gmask)
        plsc.store_scatter(outg, [gidx], gvec[blk], mask=gmask)
        lmask = lane_ok & (lrank[blk] < BEAM)
        lidx = jnp.minimum(lrank[blk], BEAM - 1)
        plsc.store_scatter(outl, [lidx], lane + blk * LANES, mask=lmask)

    out_off = pl.multiple_of(b * BEAM, 8)
    pltpu.sync_copy(outv, tv_hbm.at[pl.ds(out_off, BEAM)])
    pltpu.sync_copy(outg, ti_hbm.at[pl.ds(out_off, BEAM)])
    pltpu.sync_copy(outl, li_hbm.at[pl.ds(out_off, BEAM)])


_sc_call = functools.partial(
    pl.kernel,
    out_type=[
        jax.ShapeDtypeStruct((ROWS * BEAM,), jnp.float32),
        jax.ShapeDtypeStruct((ROWS * BEAM,), jnp.int32),
        jax.ShapeDtypeStruct((ROWS * BEAM,), jnp.int32),
    ],
    mesh=plsc.VectorSubcoreMesh(
        core_axis_name="c", subcore_axis_name="s", num_cores=2,
        num_subcores=16),
    compiler_params=pltpu.CompilerParams(needs_layout_passes=False),
    scratch_types=[
        pltpu.VMEM((2 * LANES,), jnp.int32),
        pltpu.VMEM((K + 1, 8, 128), jnp.float32),
        pltpu.VMEM((BEAM,), jnp.float32),
        pltpu.VMEM((BEAM,), jnp.int32),
        pltpu.VMEM((BEAM,), jnp.int32),
        pltpu.SemaphoreType.DMA,
    ],
)(_sc_body)


def kernel(weighted_scores, ids):
    ids_flat = ids.astype(jnp.int32).reshape(-1)
    tv, ti, li = _sc_call(weighted_scores, ids_flat)
    return (tv.reshape(ROWS, BEAM), ti.reshape(ROWS, BEAM),
            li.reshape(ROWS, BEAM))
